# baseline (speedup 1.0000x reference)
import functools

import jax
import jax.numpy as jnp
from jax import lax
from jax.experimental import pallas as pl
from jax.experimental.pallas import tpu as pltpu
from jax.experimental.pallas import tpu_sc as plsc

N_NODES = 10000
N_PAD = 10240
N_EDGES = 320000
NC = 2
NS = 16
NW = NC * NS
A_CHUNK = 128
A_ROWS = 80
S_CHUNK = 80
S_ROWS = 2 * N_EDGES // S_CHUNK // NW
NB = 4
ROWS_PER_SUB = N_PAD // NS

_mesh = functools.partial(
    plsc.VectorSubcoreMesh, core_axis_name="c", subcore_axis_name="s"
)


def _zero_fill(zv, nrow, ncol):
    for r in range(nrow):
        for j in range(ncol // 16):
            zv[r, pl.ds(j * 16, 16)] = jnp.zeros((16,), jnp.float32)


def _count_call(idx3d):
    def body(idx_hbm, out_hbm, idxb, ones_v, zv, acc, sem):
        c = lax.axis_index("c")
        s = lax.axis_index("s")
        wid = s * NC + c
        _zero_fill(zv, 8, 128)
        for r in range(A_CHUNK):
            for j in range(8):
                ones_v[r, pl.ds(j * 16, 16)] = jnp.ones((16,), jnp.float32)

        def zloop(j, _):
            pltpu.sync_copy(zv, acc.at[pl.ds(s * ROWS_PER_SUB + j * 8, 8)])
            return 0
        lax.fori_loop(0, ROWS_PER_SUB // 8, zloop, 0)
        plsc.subcore_barrier()

        pltpu.sync_copy(idx_hbm.at[wid], idxb)

        def fire(o, _):
            for b in range(8):
                pltpu.make_async_copy(
                    ones_v, acc.at[idxb.at[o * 8 + b]], sem).start(add=True)
            for b in range(8):
                pltpu.make_async_copy(
                    ones_v, acc.at[idxb.at[o * 8 + b]], sem).wait()
            return 0
        lax.fori_loop(0, A_ROWS // 8, fire, 0)

        plsc.subcore_barrier()
        pltpu.sync_copy(acc.at[pl.ds(s * ROWS_PER_SUB, ROWS_PER_SUB)],
                        out_hbm.at[pl.ds(c * N_PAD + s * ROWS_PER_SUB,
                                         ROWS_PER_SUB)])

    k = pl.kernel(
        body,
        out_type=jax.ShapeDtypeStruct((NC * N_PAD, 128), jnp.float32),
        mesh=_mesh(),
        scratch_types=[
            pltpu.VMEM((A_ROWS, A_CHUNK), jnp.int32),
            pltpu.VMEM((A_CHUNK, 128), jnp.float32),
            pltpu.VMEM((8, 128), jnp.float32),
            pltpu.VMEM_SHARED((N_PAD, 128), jnp.float32),
            pltpu.SemaphoreType.DMA,
        ],
    )
    return k(idx3d).reshape(NC, N_PAD, 128)


def _agg_call(src3d, dst3d, table, d):
    half = A_ROWS // 2

    def body(src_hbm, dst_hbm, tab_hbm, out_hbm, srcb, dstb,
             r0, r1, zv, acc, g0, g1, s0, s1):
        rows = (r0, r1)
        gs = (g0, g1)
        ss = (s0, s1)
        c = lax.axis_index("c")
        s = lax.axis_index("s")
        wid = s * NC + c
        _zero_fill(zv, 8, d)

        def zloop(j, _):
            pltpu.sync_copy(zv, acc.at[pl.ds(s * ROWS_PER_SUB + j * 8, 8)])
            return 0
        lax.fori_loop(0, ROWS_PER_SUB // 8, zloop, 0)
        plsc.subcore_barrier()

        def passloop(p, _):
            pltpu.sync_copy(src_hbm.at[wid, pl.ds(p * half, half)], srcb)
            pltpu.sync_copy(dst_hbm.at[wid, pl.ds(p * half, half)], dstb)

            for b in range(2):
                pltpu.make_async_copy(tab_hbm.at[srcb.at[b]], rows[b],
                                      gs[b]).start()

            def inner(o, _):
                for b in range(2):
                    j = o * 2 + b
                    pltpu.make_async_copy(tab_hbm.at[srcb.at[j]], rows[b],
                                          gs[b]).wait()
                    sc = pltpu.make_async_copy(rows[b], acc.at[dstb.at[j]],
                                               ss[b])
                    sc.start(add=True)
                    sc.wait()

                    @pl.when(j + 2 < half)
                    def _():
                        pltpu.make_async_copy(tab_hbm.at[srcb.at[j + 2]],
                                              rows[b], gs[b]).start()
                return 0
            lax.fori_loop(0, half // 2, inner, 0)
            return 0
        lax.fori_loop(0, 2, passloop, 0)

        plsc.subcore_barrier()
        pltpu.sync_copy(acc.at[pl.ds(s * ROWS_PER_SUB, ROWS_PER_SUB)],
                        out_hbm.at[pl.ds(c * N_PAD + s * ROWS_PER_SUB,
                                         ROWS_PER_SUB)])

    k = pl.kernel(
        body,
        out_type=jax.ShapeDtypeStruct((NC * N_PAD, d), jnp.float32),
        mesh=_mesh(),
        scratch_types=[
            pltpu.VMEM((half, A_CHUNK), jnp.int32),
            pltpu.VMEM((half, A_CHUNK), jnp.int32),
            pltpu.VMEM((A_CHUNK, d), jnp.float32),
            pltpu.VMEM((A_CHUNK, d), jnp.float32),
            pltpu.VMEM((8, d), jnp.float32),
            pltpu.VMEM_SHARED((N_PAD, d), jnp.float32),
            pltpu.SemaphoreType.DMA,
            pltpu.SemaphoreType.DMA,
            pltpu.SemaphoreType.DMA,
            pltpu.SemaphoreType.DMA,
        ],
    )
    return k(src3d, dst3d, table).reshape(NC, N_PAD, d)


def _score_call(h, u3d, v3d, n_dot):
    d = h.shape[1]
    rows_h = u3d.shape[1]
    n_out = NW * rows_h * S_CHUNK

    def body(h_hbm, u_hbm, v_hbm, out_hbm, ub, vb,
             ru0, ru1, rv0, rv1, pw0, pw1,
             gu0, gu1, gv0, gv1, ps0, ps1):
        rus = (ru0, ru1)
        rvs = (rv0, rv1)
        pws = (pw0, pw1)
        gus = (gu0, gu1)
        gvs = (gv0, gv1)
        pss = (ps0, ps1)
        c = lax.axis_index("c")
        s = lax.axis_index("s")
        wid = s * NC + c
        pltpu.sync_copy(u_hbm.at[wid], ub)
        pltpu.sync_copy(v_hbm.at[wid], vb)

        def wait_g(j, b):
            pltpu.make_async_copy(h_hbm.at[ub.at[j]], rus[b], gus[b]).wait()
            pltpu.make_async_copy(h_hbm.at[vb.at[j]], rvs[b], gvs[b]).wait()

        def start_g(j, b):
            pltpu.make_async_copy(h_hbm.at[ub.at[j]], rus[b], gus[b]).start()
            pltpu.make_async_copy(h_hbm.at[vb.at[j]], rvs[b], gvs[b]).start()

        def out_copy(j, b):
            return pltpu.make_async_copy(
                pws[b],
                out_hbm.at[pl.ds((wid * rows_h + j) * S_CHUNK, S_CHUNK)],
                pss[b])

        def compute(b):
            for r in range(S_CHUNK):
                for g in range(n_dot // 16):
                    sl = pl.ds(g * 16, 16)
                    pws[b][r, sl] = rus[b][r, sl] * rvs[b][r, sl]

        for b in range(2):
            start_g(b, b)

        def outer(o, _):
            for b in range(2):
                j = o * 2 + b
                wait_g(j, b)

                @pl.when(j >= 2)
                def _():
                    out_copy(j - 2, b).wait()

                compute(b)
                out_copy(j, b).start()

                @pl.when(j + 2 < rows_h)
                def _():
                    start_g(j + 2, b)
            return 0
        lax.fori_loop(0, (rows_h - 1) // 2, outer, 0)

        jt = rows_h - 1
        wait_g(jt, 0)
        out_copy(jt - 2, 0).wait()
        compute(0)
        out_copy(jt, 0).start()
        out_copy(jt, 0).wait()
        out_copy(jt - 1, 1).wait()

    k = pl.kernel(
        body,
        out_type=jax.ShapeDtypeStruct((n_out, n_dot), jnp.float32),
        mesh=_mesh(),
        scratch_types=(
            [pltpu.VMEM((rows_h, S_CHUNK), jnp.int32),
             pltpu.VMEM((rows_h, S_CHUNK), jnp.int32)]
            + [pltpu.VMEM((S_CHUNK, d), jnp.float32)] * 4
            + [pltpu.VMEM((S_CHUNK, n_dot), jnp.float32)] * 2
            + [pltpu.SemaphoreType.DMA] * 6
        ),
    )
    return k(h, u3d, v3d)


_TC_B = 1024


def _norm_from(deg_ref):
    dsum = deg_ref[0, :, 0:1] + deg_ref[1, :, 0:1]
    return lax.rsqrt(jnp.maximum(dsum, 1.0))


def _t0_body(x_ref, ds_ref, o_ref):
    o_ref[...] = x_ref[...] * _norm_from(ds_ref)


def _t1_body(a_ref, w_ref, b_ref, dd_ref, ds_ref, o_ref):
    a = a_ref[0] + a_ref[1]
    y = jnp.dot(a, w_ref[...], preferred_element_type=jnp.float32)
    y = jnp.maximum(y * _norm_from(dd_ref) + b_ref[...], 0.0)
    o_ref[...] = y * _norm_from(ds_ref)


def _t2_body(a_ref, w_ref, b_ref, w2_ref, dd_ref, ds_ref, o_ref):
    a = a_ref[0] + a_ref[1]
    y = jnp.dot(a, w_ref[...], preferred_element_type=jnp.float32)
    y = jnp.maximum(y * _norm_from(dd_ref) + b_ref[...], 0.0)
    y = y * _norm_from(ds_ref)
    o_ref[...] = jnp.dot(y, w2_ref[...], preferred_element_type=jnp.float32)


def _t3_body(a_ref, b_ref, dd_ref, o_ref):
    a = a_ref[0] + a_ref[1]
    o_ref[...] = a * _norm_from(dd_ref) + b_ref[...]


def _sum_body(p_ref, o_ref):
    o_ref[...] = jnp.sum(p_ref[...], axis=1, keepdims=True)


def _rows_spec(dim):
    return pl.BlockSpec((2, _TC_B, dim), lambda i: (0, i, 0))


def _full_spec(shape):
    return pl.BlockSpec(shape, lambda i: tuple(0 for _ in shape))


_DEG_SPEC = pl.BlockSpec((2, _TC_B, 128), lambda i: (0, i, 0))


def _tc_call(body_fn, in_specs, out_dim, args):
    return pl.pallas_call(
        body_fn,
        grid=(N_PAD // _TC_B,),
        in_specs=in_specs,
        out_specs=pl.BlockSpec((_TC_B, out_dim), lambda i: (i, 0)),
        out_shape=jax.ShapeDtypeStruct((N_PAD, out_dim), jnp.float32),
    )(*args)


def _pad_edges(x):
    x2 = x.reshape(NW, N_EDGES // NW)
    x2 = jnp.pad(x2, ((0, 0), (0, A_ROWS * A_CHUNK - N_EDGES // NW)),
                 constant_values=N_NODES)
    return x2.reshape(NW, A_ROWS, A_CHUNK)


def kernel(features, pos_edge_index, neg_edge_index, W0, b0, W1, b1, W2, b2):
    src3d = _pad_edges(pos_edge_index[0])
    dst3d = _pad_edges(pos_edge_index[1])

    degs = _count_call(src3d)
    degd = _count_call(dst3d)

    feats_p = jnp.pad(features, ((0, N_PAD - N_NODES), (0, 0)))
    x0 = _tc_call(
        _t0_body,
        [pl.BlockSpec((_TC_B, 128), lambda i: (i, 0)), _DEG_SPEC],
        128, (feats_p, degs))
    a0 = _agg_call(src3d, dst3d, x0, 128)

    x1 = _tc_call(
        _t1_body,
        [_rows_spec(128), _full_spec((128, 128)), _full_spec((1, 128)),
         _DEG_SPEC, _DEG_SPEC],
        128, (a0, W0, b0.reshape(1, 128), degd, degs))
    a1 = _agg_call(src3d, dst3d, x1, 128)

    W2p = jnp.pad(W2, ((0, 0), (0, 64)))
    x2 = _tc_call(
        _t2_body,
        [_rows_spec(128), _full_spec((128, 128)), _full_spec((1, 128)),
         _full_spec((128, 128)), _DEG_SPEC, _DEG_SPEC],
        128, (a1, W1, b1.reshape(1, 128), W2p, degd, degs))
    a2 = _agg_call(src3d, dst3d, x2, 128)

    b2p = jnp.pad(b2.reshape(1, 64), ((0, 0), (0, 64)))
    h3 = _tc_call(
        _t3_body,
        [_rows_spec(128), _full_spec((1, 128)), _DEG_SPEC],
        128, (a2, b2p, degd))

    rows_h = N_EDGES // S_CHUNK // NW
    blk = 5000

    def half_scores(u, v):
        prods = _score_call(h3, u.reshape(NW, rows_h, S_CHUNK),
                            v.reshape(NW, rows_h, S_CHUNK), 64)
        return pl.pallas_call(
            _sum_body,
            grid=(N_EDGES // blk,),
            in_specs=[pl.BlockSpec((blk, 64), lambda i: (i, 0))],
            out_specs=pl.BlockSpec((blk, 1), lambda i: (i, 0)),
            out_shape=jax.ShapeDtypeStruct((N_EDGES, 1), jnp.float32),
        )(prods)

    pos_s = half_scores(pos_edge_index[0], pos_edge_index[1])
    neg_s = half_scores(neg_edge_index[0], neg_edge_index[1])
    return jnp.concatenate([pos_s, neg_s], axis=0)


# speedup vs baseline: 1.5710x; 1.5710x over previous
import functools

import jax
import jax.numpy as jnp
from jax import lax
from jax.experimental import pallas as pl
from jax.experimental.pallas import tpu as pltpu
from jax.experimental.pallas import tpu_sc as plsc

N_NODES = 10000
N_PAD = 10240
N_EDGES = 320000
NC = 2
NS = 16
NW = NC * NS
A_CHUNK = 128
A_ROWS = 80
S_CHUNK = 80
S_ROWS = 2 * N_EDGES // S_CHUNK // NW
NB = 4
ROWS_PER_SUB = N_PAD // NS

_mesh = functools.partial(
    plsc.VectorSubcoreMesh, core_axis_name="c", subcore_axis_name="s"
)


def _zero_fill(zv, nrow, ncol):
    for r in range(nrow):
        for j in range(ncol // 16):
            zv[r, pl.ds(j * 16, 16)] = jnp.zeros((16,), jnp.float32)


def _count_call(idx3d):
    def body(idx_hbm, out_hbm, idxb, ones_v, zv, acc, sem):
        c = lax.axis_index("c")
        s = lax.axis_index("s")
        wid = s * NC + c
        _zero_fill(zv, 8, 128)
        for r in range(A_CHUNK):
            for j in range(8):
                ones_v[r, pl.ds(j * 16, 16)] = jnp.ones((16,), jnp.float32)

        def zloop(j, _):
            pltpu.sync_copy(zv, acc.at[pl.ds(s * ROWS_PER_SUB + j * 8, 8)])
            return 0
        lax.fori_loop(0, ROWS_PER_SUB // 8, zloop, 0)
        plsc.subcore_barrier()

        pltpu.sync_copy(idx_hbm.at[wid], idxb)

        def fire(o, _):
            for b in range(8):
                pltpu.make_async_copy(
                    ones_v, acc.at[idxb.at[o * 8 + b]], sem).start(add=True)
            for b in range(8):
                pltpu.make_async_copy(
                    ones_v, acc.at[idxb.at[o * 8 + b]], sem).wait()
            return 0
        lax.fori_loop(0, A_ROWS // 8, fire, 0)

        plsc.subcore_barrier()
        pltpu.sync_copy(acc.at[pl.ds(s * ROWS_PER_SUB, ROWS_PER_SUB)],
                        out_hbm.at[pl.ds(c * N_PAD + s * ROWS_PER_SUB,
                                         ROWS_PER_SUB)])

    k = pl.kernel(
        body,
        out_type=jax.ShapeDtypeStruct((NC * N_PAD, 128), jnp.float32),
        mesh=_mesh(),
        scratch_types=[
            pltpu.VMEM((A_ROWS, A_CHUNK), jnp.int32),
            pltpu.VMEM((A_CHUNK, 128), jnp.float32),
            pltpu.VMEM((8, 128), jnp.float32),
            pltpu.VMEM_SHARED((N_PAD, 128), jnp.float32),
            pltpu.SemaphoreType.DMA,
        ],
    )
    return k(idx3d).reshape(NC, N_PAD, 128)


def _agg_call(src3d, dst3d, table, d):
    half = A_ROWS // 2

    def body(src_hbm, dst_hbm, tab_hbm, out_hbm, srcb, dstb,
             r0, r1, zv, acc, g0, g1, s0, s1):
        rows = (r0, r1)
        gs = (g0, g1)
        ss = (s0, s1)
        c = lax.axis_index("c")
        s = lax.axis_index("s")
        wid = s * NC + c
        _zero_fill(zv, 8, d)

        def zloop(j, _):
            pltpu.sync_copy(zv, acc.at[pl.ds(s * ROWS_PER_SUB + j * 8, 8)])
            return 0
        lax.fori_loop(0, ROWS_PER_SUB // 8, zloop, 0)
        plsc.subcore_barrier()

        def passloop(p, _):
            pltpu.sync_copy(src_hbm.at[wid, pl.ds(p * half, half)], srcb)
            pltpu.sync_copy(dst_hbm.at[wid, pl.ds(p * half, half)], dstb)

            for b in range(2):
                pltpu.make_async_copy(tab_hbm.at[srcb.at[b]], rows[b],
                                      gs[b]).start()

            def inner(o, _):
                for b in range(2):
                    j = o * 2 + b
                    pltpu.make_async_copy(tab_hbm.at[srcb.at[j]], rows[b],
                                          gs[b]).wait()
                    sc = pltpu.make_async_copy(rows[b], acc.at[dstb.at[j]],
                                               ss[b])
                    sc.start(add=True)
                    sc.wait()

                    @pl.when(j + 2 < half)
                    def _():
                        pltpu.make_async_copy(tab_hbm.at[srcb.at[j + 2]],
                                              rows[b], gs[b]).start()
                return 0
            lax.fori_loop(0, half // 2, inner, 0)
            return 0
        lax.fori_loop(0, 2, passloop, 0)

        plsc.subcore_barrier()
        pltpu.sync_copy(acc.at[pl.ds(s * ROWS_PER_SUB, ROWS_PER_SUB)],
                        out_hbm.at[pl.ds(c * N_PAD + s * ROWS_PER_SUB,
                                         ROWS_PER_SUB)])

    k = pl.kernel(
        body,
        out_type=jax.ShapeDtypeStruct((NC * N_PAD, d), jnp.float32),
        mesh=_mesh(),
        scratch_types=[
            pltpu.VMEM((half, A_CHUNK), jnp.int32),
            pltpu.VMEM((half, A_CHUNK), jnp.int32),
            pltpu.VMEM((A_CHUNK, d), jnp.float32),
            pltpu.VMEM((A_CHUNK, d), jnp.float32),
            pltpu.VMEM((8, d), jnp.float32),
            pltpu.VMEM_SHARED((N_PAD, d), jnp.float32),
            pltpu.SemaphoreType.DMA,
            pltpu.SemaphoreType.DMA,
            pltpu.SemaphoreType.DMA,
            pltpu.SemaphoreType.DMA,
        ],
    )
    return k(src3d, dst3d, table).reshape(NC, N_PAD, d)


def _score_call(h, u3d, v3d, n_dot):
    d = h.shape[1]
    rows_h = u3d.shape[1]
    n_out = NW * rows_h * S_CHUNK

    def body(h_hbm, u_hbm, v_hbm, out_hbm, ub, vb,
             ru0, ru1, rv0, rv1, pw0, pw1,
             gu0, gu1, gv0, gv1, ps0, ps1):
        rus = (ru0, ru1)
        rvs = (rv0, rv1)
        pws = (pw0, pw1)
        gus = (gu0, gu1)
        gvs = (gv0, gv1)
        pss = (ps0, ps1)
        c = lax.axis_index("c")
        s = lax.axis_index("s")
        wid = s * NC + c
        pltpu.sync_copy(u_hbm.at[wid], ub)
        pltpu.sync_copy(v_hbm.at[wid], vb)

        def wait_g(j, b):
            pltpu.make_async_copy(h_hbm.at[ub.at[j]], rus[b], gus[b]).wait()
            pltpu.make_async_copy(h_hbm.at[vb.at[j]], rvs[b], gvs[b]).wait()

        def start_g(j, b):
            pltpu.make_async_copy(h_hbm.at[ub.at[j]], rus[b], gus[b]).start()
            pltpu.make_async_copy(h_hbm.at[vb.at[j]], rvs[b], gvs[b]).start()

        def out_copy(j, b):
            return pltpu.make_async_copy(
                pws[b],
                out_hbm.at[pl.ds((wid * rows_h + j) * S_CHUNK, S_CHUNK)],
                pss[b])

        def compute(b):
            for r in range(S_CHUNK):
                for g in range(n_dot // 16):
                    sl = pl.ds(g * 16, 16)
                    pws[b][r, sl] = rus[b][r, sl] * rvs[b][r, sl]

        for b in range(2):
            start_g(b, b)

        def outer(o, _):
            for b in range(2):
                j = o * 2 + b
                wait_g(j, b)

                @pl.when(j >= 2)
                def _():
                    out_copy(j - 2, b).wait()

                compute(b)
                out_copy(j, b).start()

                @pl.when(j + 2 < rows_h)
                def _():
                    start_g(j + 2, b)
            return 0
        lax.fori_loop(0, (rows_h - 1) // 2, outer, 0)

        jt = rows_h - 1
        wait_g(jt, 0)
        out_copy(jt - 2, 0).wait()
        compute(0)
        out_copy(jt, 0).start()
        out_copy(jt, 0).wait()
        out_copy(jt - 1, 1).wait()

    k = pl.kernel(
        body,
        out_type=jax.ShapeDtypeStruct((n_out, n_dot), jnp.float32),
        mesh=_mesh(),
        scratch_types=(
            [pltpu.VMEM((rows_h, S_CHUNK), jnp.int32),
             pltpu.VMEM((rows_h, S_CHUNK), jnp.int32)]
            + [pltpu.VMEM((S_CHUNK, d), jnp.float32)] * 4
            + [pltpu.VMEM((S_CHUNK, n_dot), jnp.float32)] * 2
            + [pltpu.SemaphoreType.DMA] * 6
        ),
    )
    return k(h, u3d, v3d)


_TC_B = 1024


def _norm_from(deg_ref):
    dsum = deg_ref[0, :, 0:1] + deg_ref[1, :, 0:1]
    return lax.rsqrt(jnp.maximum(dsum, 1.0))


def _t0_body(x_ref, ds_ref, o_ref):
    o_ref[...] = x_ref[...] * _norm_from(ds_ref)


def _t1_body(a_ref, w_ref, b_ref, dd_ref, ds_ref, o_ref):
    a = a_ref[0] + a_ref[1]
    y = jnp.dot(a, w_ref[...], preferred_element_type=jnp.float32)
    y = jnp.maximum(y * _norm_from(dd_ref) + b_ref[...], 0.0)
    o_ref[...] = y * _norm_from(ds_ref)


def _t2_body(a_ref, w_ref, b_ref, w2_ref, dd_ref, ds_ref, o_ref):
    a = a_ref[0] + a_ref[1]
    y = jnp.dot(a, w_ref[...], preferred_element_type=jnp.float32)
    y = jnp.maximum(y * _norm_from(dd_ref) + b_ref[...], 0.0)
    y = y * _norm_from(ds_ref)
    o_ref[...] = jnp.dot(y, w2_ref[...], preferred_element_type=jnp.float32)


def _t3_body(a_ref, b_ref, dd_ref, o_ref):
    a = a_ref[0] + a_ref[1]
    o_ref[...] = a * _norm_from(dd_ref) + b_ref[...]


def _sum_body(p_ref, o_ref):
    o_ref[...] = jnp.sum(p_ref[...], axis=1, keepdims=True)


def _rows_spec(dim):
    return pl.BlockSpec((2, _TC_B, dim), lambda i: (0, i, 0))


def _full_spec(shape):
    return pl.BlockSpec(shape, lambda i: tuple(0 for _ in shape))


_DEG_SPEC = pl.BlockSpec((2, _TC_B, 128), lambda i: (0, i, 0))


def _tc_call(body_fn, in_specs, out_dim, args):
    return pl.pallas_call(
        body_fn,
        grid=(N_PAD // _TC_B,),
        in_specs=in_specs,
        out_specs=pl.BlockSpec((_TC_B, out_dim), lambda i: (i, 0)),
        out_shape=jax.ShapeDtypeStruct((N_PAD, out_dim), jnp.float32),
    )(*args)


def _pad_edges(x):
    x2 = x.reshape(NW, N_EDGES // NW)
    npad = A_ROWS * A_CHUNK - N_EDGES // NW
    padv = jnp.arange(NW, dtype=jnp.int32)[:, None] + N_NODES
    pad_block = jnp.broadcast_to(padv, (NW, npad))
    return jnp.concatenate([x2, pad_block], axis=1).reshape(
        NW, A_ROWS, A_CHUNK)


def kernel(features, pos_edge_index, neg_edge_index, W0, b0, W1, b1, W2, b2):
    src3d = _pad_edges(pos_edge_index[0])
    dst3d = _pad_edges(pos_edge_index[1])

    degs = _count_call(src3d)
    degd = _count_call(dst3d)

    feats_p = jnp.pad(features, ((0, N_PAD - N_NODES), (0, 0)))
    x0 = _tc_call(
        _t0_body,
        [pl.BlockSpec((_TC_B, 128), lambda i: (i, 0)), _DEG_SPEC],
        128, (feats_p, degs))
    a0 = _agg_call(src3d, dst3d, x0, 128)

    x1 = _tc_call(
        _t1_body,
        [_rows_spec(128), _full_spec((128, 128)), _full_spec((1, 128)),
         _DEG_SPEC, _DEG_SPEC],
        128, (a0, W0, b0.reshape(1, 128), degd, degs))
    a1 = _agg_call(src3d, dst3d, x1, 128)

    W2p = jnp.pad(W2, ((0, 0), (0, 64)))
    x2 = _tc_call(
        _t2_body,
        [_rows_spec(128), _full_spec((128, 128)), _full_spec((1, 128)),
         _full_spec((128, 128)), _DEG_SPEC, _DEG_SPEC],
        128, (a1, W1, b1.reshape(1, 128), W2p, degd, degs))
    a2 = _agg_call(src3d, dst3d, x2, 128)

    b2p = jnp.pad(b2.reshape(1, 64), ((0, 0), (0, 64)))
    h3 = _tc_call(
        _t3_body,
        [_rows_spec(128), _full_spec((1, 128)), _DEG_SPEC],
        128, (a2, b2p, degd))

    rows_h = N_EDGES // S_CHUNK // NW
    blk = 5000

    def half_scores(u, v):
        prods = _score_call(h3, u.reshape(NW, rows_h, S_CHUNK),
                            v.reshape(NW, rows_h, S_CHUNK), 64)
        return pl.pallas_call(
            _sum_body,
            grid=(N_EDGES // blk,),
            in_specs=[pl.BlockSpec((blk, 64), lambda i: (i, 0))],
            out_specs=pl.BlockSpec((blk, 1), lambda i: (i, 0)),
            out_shape=jax.ShapeDtypeStruct((N_EDGES, 1), jnp.float32),
        )(prods)

    pos_s = half_scores(pos_edge_index[0], pos_edge_index[1])
    neg_s = half_scores(neg_edge_index[0], neg_edge_index[1])
    return jnp.concatenate([pos_s, neg_s], axis=0)
